# baseline (speedup 1.0000x reference)
import functools

import jax
import jax.numpy as jnp
from jax import lax
from jax.experimental import pallas as pl
from jax.experimental.pallas import tpu as pltpu
from jax.experimental.pallas import tpu_sc as plsc

NTOKEN = 100000
D = 64
B = 16384
L = 50
N = B * L
NC = 2
NS = 16
NW = NC * NS
CHUNK = 128
PAIRS = CHUNK // 2
PER_W = N // NW
NSTEPS = PER_W // CHUNK
LANES = 16

_mesh = plsc.VectorSubcoreMesh(core_axis_name="c", subcore_axis_name="s")


@functools.partial(
    pl.kernel,
    out_type=(
        jax.ShapeDtypeStruct((N // 2, 128), jnp.float32),
        jax.ShapeDtypeStruct((N // 2, 128), jnp.float32),
        jax.ShapeDtypeStruct((NW * LANES,), jnp.float32),
    ),
    mesh=_mesh,
    compiler_params=pltpu.CompilerParams(use_tc_tiling_on_sc=False),
    scratch_types=[
        pltpu.VMEM((NSTEPS, CHUNK), jnp.int32),
        [pltpu.VMEM((CHUNK, D), jnp.float32)] * 2,
        [pltpu.VMEM((CHUNK, D), jnp.float32)] * 2,
        [pltpu.VMEM((PAIRS, 128), jnp.float32)] * 2,
        [pltpu.VMEM((PAIRS, 128), jnp.float32)] * 2,
        pltpu.VMEM((LANES,), jnp.float32),
        [pltpu.SemaphoreType.DMA] * 2,
        [pltpu.SemaphoreType.DMA] * 2,
    ],
)
def _sc_gather(idx_hbm, enc_hbm, glv_hbm, out_e, out_g, out_p,
               idx_v, e_slots, g_slots, es_slots, gs_slots, acc_v, gsems, wsems):
    wid = lax.axis_index("s") * NC + lax.axis_index("c")
    row0 = wid * PER_W
    q0 = row0 // 2
    pltpu.sync_copy(idx_hbm.at[pl.ds(wid * NSTEPS, NSTEPS)], idx_v)

    def fire_gather(k, s):
        pltpu.async_copy(enc_hbm.at[idx_v.at[k]], e_slots[s], gsems[s])
        pltpu.async_copy(glv_hbm.at[idx_v.at[k]], g_slots[s], gsems[s])

    def wait_gather(s):
        pltpu.make_async_copy(enc_hbm.at[idx_v.at[0]], e_slots[s], gsems[s]).wait()
        pltpu.make_async_copy(glv_hbm.at[idx_v.at[0]], g_slots[s], gsems[s]).wait()

    def fire_write(k, s):
        dst = pl.ds(q0 + k * PAIRS, PAIRS)
        pltpu.async_copy(es_slots[s], out_e.at[dst], wsems[s])
        pltpu.async_copy(gs_slots[s], out_g.at[dst], wsems[s])

    def wait_write(s):
        pltpu.make_async_copy(es_slots[s], out_e.at[pl.ds(0, PAIRS)], wsems[s]).wait()
        pltpu.make_async_copy(gs_slots[s], out_g.at[pl.ds(0, PAIRS)], wsems[s]).wait()

    def compute(s, accs):
        e_v, g_v = e_slots[s], g_slots[s]
        es_v, gs_v = es_slots[s], gs_slots[s]

        def pair_body(p, accs):
            a0, a1, a2, a3 = accs
            i0 = 2 * p
            i1 = i0 + 1
            for j in range(4):
                ve = e_v[i0, pl.ds(16 * j, LANES)]
                vg = g_v[i0, pl.ds(16 * j, LANES)]
                es_v[p, pl.ds(16 * j, LANES)] = ve
                gs_v[p, pl.ds(16 * j, LANES)] = vg
                d = ve - vg
                if j == 0:
                    a0 += d * d
                elif j == 1:
                    a1 += d * d
                elif j == 2:
                    a2 += d * d
                else:
                    a3 += d * d
            for j in range(4):
                ve = e_v[i1, pl.ds(16 * j, LANES)]
                vg = g_v[i1, pl.ds(16 * j, LANES)]
                es_v[p, pl.ds(64 + 16 * j, LANES)] = ve
                gs_v[p, pl.ds(64 + 16 * j, LANES)] = vg
                d = ve - vg
                if j == 0:
                    a0 += d * d
                elif j == 1:
                    a1 += d * d
                elif j == 2:
                    a2 += d * d
                else:
                    a3 += d * d
            return (a0, a1, a2, a3)

        return lax.fori_loop(0, PAIRS, pair_body, accs)

    def service(k, s, accs, *, first=False, last=False):
        wait_gather(s)
        if not first:
            wait_write(s)
        accs = compute(s, accs)
        fire_write(k, s)
        if not last:
            fire_gather(k + 2, s)
        return accs

    zero = jnp.zeros((LANES,), jnp.float32)
    accs = (zero, zero, zero, zero)

    fire_gather(0, 0)
    fire_gather(1, 1)
    accs = service(0, 0, accs, first=True)
    accs = service(1, 1, accs, first=True)

    def group_body(g, accs):
        k = 2 * g
        accs = service(k, 0, accs)
        accs = service(k + 1, 1, accs)
        return accs

    accs = lax.fori_loop(1, NSTEPS // 2 - 1, group_body, accs)

    k = NSTEPS - 2
    accs = service(k, 0, accs, last=True)
    accs = service(k + 1, 1, accs, last=True)
    wait_write(0)
    wait_write(1)

    a0, a1, a2, a3 = accs
    acc_v[...] = (a0 + a1) + (a2 + a3)
    pltpu.sync_copy(acc_v, out_p.at[pl.ds(wid * LANES, LANES)])


def _tc_sum_body(p_ref, o_ref):
    o_ref[0, 0] = jnp.sum(p_ref[...]) * jnp.float32(1.0 / (N * D))


_tc_sum = pl.pallas_call(
    _tc_sum_body,
    out_shape=jax.ShapeDtypeStruct((1, 1), jnp.float32),
    out_specs=pl.BlockSpec(memory_space=pltpu.SMEM),
)

_BB = 16
_ROWS = _BB * L * D // 128


def _unpair(x):
    lo = x[:, None, :D]
    hi = x[:, None, D:]
    return jnp.concatenate([lo, hi], axis=1).reshape(_BB, L, D)


def _tc_fix_body(e_ref, g_ref, oe_ref, og_ref):
    oe_ref[...] = _unpair(e_ref[...])
    og_ref[...] = _unpair(g_ref[...])


_tc_fix = pl.pallas_call(
    _tc_fix_body,
    grid=(B // _BB,),
    in_specs=[pl.BlockSpec((_ROWS, 128), lambda i: (i, 0)),
              pl.BlockSpec((_ROWS, 128), lambda i: (i, 0))],
    out_specs=[pl.BlockSpec((_BB, L, D), lambda i: (i, 0, 0)),
               pl.BlockSpec((_BB, L, D), lambda i: (i, 0, 0))],
    out_shape=[jax.ShapeDtypeStruct((B, L, D), jnp.float32),
               jax.ShapeDtypeStruct((B, L, D), jnp.float32)],
)


def kernel(input, encoder_weight, glove_weight):
    idx = input.reshape(N // CHUNK, CHUNK).astype(jnp.int32)
    emb2, emb_glove2, parts = _sc_gather(idx, encoder_weight, glove_weight)
    glove_loss = _tc_sum(parts.reshape(4, 128))[0, 0]
    emb, emb_glove = _tc_fix(emb2, emb_glove2)
    return (emb, emb_glove, glove_loss)


# speedup vs baseline: 1.3386x; 1.3386x over previous
import functools

import jax
import jax.numpy as jnp
from jax import lax
from jax.experimental import pallas as pl
from jax.experimental.pallas import tpu as pltpu
from jax.experimental.pallas import tpu_sc as plsc

NTOKEN = 100000
D = 64
B = 16384
L = 50
N = B * L
NC = 2
NS = 16
NW = NC * NS
CHUNK = 128
PAIRS = CHUNK // 2
PER_W = N // NW
NSTEPS = PER_W // CHUNK
LANES = 16

_mesh = plsc.VectorSubcoreMesh(core_axis_name="c", subcore_axis_name="s")


@functools.partial(
    pl.kernel,
    out_type=(
        jax.ShapeDtypeStruct((N // 2, 128), jnp.float32),
        jax.ShapeDtypeStruct((N // 2, 128), jnp.float32),
        jax.ShapeDtypeStruct((NW * LANES,), jnp.float32),
    ),
    mesh=_mesh,
    compiler_params=pltpu.CompilerParams(use_tc_tiling_on_sc=False),
    scratch_types=[
        pltpu.VMEM((NSTEPS, CHUNK), jnp.int32),
        [pltpu.VMEM((CHUNK, D), jnp.float32)] * 2,
        [pltpu.VMEM((CHUNK, D), jnp.float32)] * 2,
        [pltpu.VMEM((PAIRS, 128), jnp.float32)] * 2,
        [pltpu.VMEM((PAIRS, 128), jnp.float32)] * 2,
        pltpu.VMEM((LANES,), jnp.float32),
        [pltpu.SemaphoreType.DMA] * 2,
        [pltpu.SemaphoreType.DMA] * 2,
    ],
)
def _sc_gather(idx_hbm, enc_hbm, glv_hbm, out_e, out_g, out_p,
               idx_v, e_slots, g_slots, es_slots, gs_slots, acc_v, gsems, wsems):
    wid = lax.axis_index("s") * NC + lax.axis_index("c")
    row0 = wid * PER_W
    q0 = row0 // 2
    pltpu.sync_copy(idx_hbm.at[pl.ds(wid * NSTEPS, NSTEPS)], idx_v)

    def fire_gather(k, s):
        pltpu.async_copy(enc_hbm.at[idx_v.at[k]], e_slots[s], gsems[s])
        pltpu.async_copy(glv_hbm.at[idx_v.at[k]], g_slots[s], gsems[s])

    def wait_gather(s):
        pltpu.make_async_copy(enc_hbm.at[idx_v.at[0]], e_slots[s], gsems[s]).wait()
        pltpu.make_async_copy(glv_hbm.at[idx_v.at[0]], g_slots[s], gsems[s]).wait()

    def fire_write(k, s):
        dst = pl.ds(q0 + k * PAIRS, PAIRS)
        pltpu.async_copy(es_slots[s], out_e.at[dst], wsems[s])
        pltpu.async_copy(gs_slots[s], out_g.at[dst], wsems[s])

    def wait_write(s):
        pltpu.make_async_copy(es_slots[s], out_e.at[pl.ds(0, PAIRS)], wsems[s]).wait()
        pltpu.make_async_copy(gs_slots[s], out_g.at[pl.ds(0, PAIRS)], wsems[s]).wait()

    def compute(s, accs):
        e_v, g_v = e_slots[s], g_slots[s]
        es_v, gs_v = es_slots[s], gs_slots[s]

        def pair_body(p, accs):
            a0, a1, a2, a3 = accs
            i0 = 2 * p
            i1 = i0 + 1
            for j in range(4):
                ve = e_v[i0, pl.ds(16 * j, LANES)]
                vg = g_v[i0, pl.ds(16 * j, LANES)]
                es_v[p, pl.ds(16 * j, LANES)] = ve
                gs_v[p, pl.ds(16 * j, LANES)] = vg
                d = ve - vg
                if j == 0:
                    a0 += d * d
                elif j == 1:
                    a1 += d * d
                elif j == 2:
                    a2 += d * d
                else:
                    a3 += d * d
            for j in range(4):
                ve = e_v[i1, pl.ds(16 * j, LANES)]
                vg = g_v[i1, pl.ds(16 * j, LANES)]
                es_v[p, pl.ds(64 + 16 * j, LANES)] = ve
                gs_v[p, pl.ds(64 + 16 * j, LANES)] = vg
                d = ve - vg
                if j == 0:
                    a0 += d * d
                elif j == 1:
                    a1 += d * d
                elif j == 2:
                    a2 += d * d
                else:
                    a3 += d * d
            return (a0, a1, a2, a3)

        return lax.fori_loop(0, PAIRS, pair_body, accs)

    def service(k, s, accs, *, first=False, last=False):
        wait_gather(s)
        if not first:
            wait_write(s)
        accs = compute(s, accs)
        fire_write(k, s)
        if not last:
            fire_gather(k + 2, s)
        return accs

    zero = jnp.zeros((LANES,), jnp.float32)
    accs = (zero, zero, zero, zero)

    fire_gather(0, 0)
    fire_gather(1, 1)
    accs = service(0, 0, accs, first=True)
    accs = service(1, 1, accs, first=True)

    def group_body(g, accs):
        k = 2 * g
        accs = service(k, 0, accs)
        accs = service(k + 1, 1, accs)
        return accs

    accs = lax.fori_loop(1, NSTEPS // 2 - 1, group_body, accs)

    k = NSTEPS - 2
    accs = service(k, 0, accs, last=True)
    accs = service(k + 1, 1, accs, last=True)
    wait_write(0)
    wait_write(1)

    a0, a1, a2, a3 = accs
    acc_v[...] = (a0 + a1) + (a2 + a3)
    pltpu.sync_copy(acc_v, out_p.at[pl.ds(wid * LANES, LANES)])


def _tc_sum_body(p_ref, o_ref):
    o_ref[0, 0] = jnp.sum(p_ref[...]) * jnp.float32(1.0 / (N * D))


_tc_sum = pl.pallas_call(
    _tc_sum_body,
    out_shape=jax.ShapeDtypeStruct((1, 1), jnp.float32),
    out_specs=pl.BlockSpec(memory_space=pltpu.SMEM),
)

def kernel(input, encoder_weight, glove_weight):
    idx = input.reshape(N // CHUNK, CHUNK).astype(jnp.int32)
    emb2, emb_glove2, parts = _sc_gather(idx, encoder_weight, glove_weight)
    glove_loss = _tc_sum(parts.reshape(4, 128))[0, 0]
    keep_on_tc = 0.0 * glove_loss
    emb = emb2.reshape(B, L, D) + keep_on_tc
    emb_glove = emb_glove2.reshape(B, L, D) + keep_on_tc
    return (emb, emb_glove, glove_loss)


# speedup vs baseline: 1.4539x; 1.0862x over previous
import functools

import jax
import jax.numpy as jnp
from jax import lax
from jax.experimental import pallas as pl
from jax.experimental.pallas import tpu as pltpu
from jax.experimental.pallas import tpu_sc as plsc

NTOKEN = 100000
D = 64
B = 16384
L = 50
N = B * L
NC = 2
NS = 16
NW = NC * NS
CHUNK = 128
PAIRS = CHUNK // 2
PER_W = N // NW
NSTEPS = PER_W // CHUNK
LANES = 16

_mesh = plsc.VectorSubcoreMesh(core_axis_name="c", subcore_axis_name="s")


@functools.partial(
    pl.kernel,
    out_type=(
        jax.ShapeDtypeStruct((N // 2, 128), jnp.float32),
        jax.ShapeDtypeStruct((N // 2, 128), jnp.float32),
        jax.ShapeDtypeStruct((NW * LANES,), jnp.float32),
    ),
    mesh=_mesh,
    compiler_params=pltpu.CompilerParams(use_tc_tiling_on_sc=False),
    scratch_types=[
        pltpu.VMEM((NSTEPS, CHUNK), jnp.int32),
        [pltpu.VMEM((CHUNK, D), jnp.float32)] * 2,
        [pltpu.VMEM((CHUNK, D), jnp.float32)] * 2,
        [pltpu.VMEM((PAIRS, 128), jnp.float32)] * 2,
        [pltpu.VMEM((PAIRS, 128), jnp.float32)] * 2,
        pltpu.VMEM((LANES,), jnp.float32),
        [pltpu.SemaphoreType.DMA] * 2,
        [pltpu.SemaphoreType.DMA] * 2,
    ],
)
def _sc_gather(idx_hbm, enc_hbm, glv_hbm, out_e, out_g, out_p,
               idx_v, e_slots, g_slots, es_slots, gs_slots, acc_v, gsems, wsems):
    wid = lax.axis_index("s") * NC + lax.axis_index("c")
    row0 = wid * PER_W
    q0 = row0 // 2
    pltpu.sync_copy(idx_hbm.at[pl.ds(wid * NSTEPS, NSTEPS)], idx_v)

    def fire_gather(k, s):
        pltpu.async_copy(enc_hbm.at[idx_v.at[k]], e_slots[s], gsems[s])
        pltpu.async_copy(glv_hbm.at[idx_v.at[k]], g_slots[s], gsems[s])

    def wait_gather(s):
        pltpu.make_async_copy(enc_hbm.at[idx_v.at[0]], e_slots[s], gsems[s]).wait()
        pltpu.make_async_copy(glv_hbm.at[idx_v.at[0]], g_slots[s], gsems[s]).wait()

    def fire_write(k, s):
        dst = pl.ds(q0 + k * PAIRS, PAIRS)
        pltpu.async_copy(es_slots[s], out_e.at[dst], wsems[s])
        pltpu.async_copy(gs_slots[s], out_g.at[dst], wsems[s])

    def wait_write(s):
        pltpu.make_async_copy(es_slots[s], out_e.at[pl.ds(0, PAIRS)], wsems[s]).wait()
        pltpu.make_async_copy(gs_slots[s], out_g.at[pl.ds(0, PAIRS)], wsems[s]).wait()

    def compute(s, accs):
        e_v, g_v = e_slots[s], g_slots[s]
        es_v, gs_v = es_slots[s], gs_slots[s]

        def pair_body(p, accs):
            a0, a1, a2, a3 = accs
            i0 = 2 * p
            i1 = i0 + 1
            for j in range(4):
                ve = e_v[i0, pl.ds(16 * j, LANES)]
                vg = g_v[i0, pl.ds(16 * j, LANES)]
                es_v[p, pl.ds(16 * j, LANES)] = ve
                gs_v[p, pl.ds(16 * j, LANES)] = vg
                d = ve - vg
                if j == 0:
                    a0 += d * d
                elif j == 1:
                    a1 += d * d
                elif j == 2:
                    a2 += d * d
                else:
                    a3 += d * d
            for j in range(4):
                ve = e_v[i1, pl.ds(16 * j, LANES)]
                vg = g_v[i1, pl.ds(16 * j, LANES)]
                es_v[p, pl.ds(64 + 16 * j, LANES)] = ve
                gs_v[p, pl.ds(64 + 16 * j, LANES)] = vg
                d = ve - vg
                if j == 0:
                    a0 += d * d
                elif j == 1:
                    a1 += d * d
                elif j == 2:
                    a2 += d * d
                else:
                    a3 += d * d
            return (a0, a1, a2, a3)

        return lax.fori_loop(0, PAIRS, pair_body, accs)

    def service(k, s, accs, *, first=False, last=False):
        wait_gather(s)
        if not first:
            wait_write(s)
        accs = compute(s, accs)
        fire_write(k, s)
        if not last:
            fire_gather(k + 2, s)
        return accs

    zero = jnp.zeros((LANES,), jnp.float32)
    accs = (zero, zero, zero, zero)

    fire_gather(0, 0)
    fire_gather(1, 1)
    accs = service(0, 0, accs, first=True)
    accs = service(1, 1, accs, first=True)

    def group_body(g, accs):
        k = 2 * g
        accs = service(k, 0, accs)
        accs = service(k + 1, 1, accs)
        return accs

    accs = lax.fori_loop(1, NSTEPS // 2 - 1, group_body, accs)

    k = NSTEPS - 2
    accs = service(k, 0, accs, last=True)
    accs = service(k + 1, 1, accs, last=True)
    wait_write(0)
    wait_write(1)

    a0, a1, a2, a3 = accs
    acc_v[...] = (a0 + a1) + (a2 + a3)
    pltpu.sync_copy(acc_v, out_p.at[pl.ds(wid * LANES, LANES)])


def _tc_sum_body(p_ref, o_ref):
    o_ref[0, 0] = jnp.sum(p_ref[...]) * jnp.float32(1.0 / (N * D))


_tc_sum = pl.pallas_call(
    _tc_sum_body,
    out_shape=jax.ShapeDtypeStruct((1, 1), jnp.float32),
    out_specs=pl.BlockSpec(memory_space=pltpu.SMEM),
)

def kernel(input, encoder_weight, glove_weight):
    idx = input.reshape(N // CHUNK, CHUNK).astype(jnp.int32)
    emb2, emb_glove2, parts = _sc_gather(idx, encoder_weight, glove_weight)
    glove_loss = _tc_sum(parts.reshape(4, 128))[0, 0]
    emb = emb2.reshape(B, L, D)
    emb_glove = emb_glove2.reshape(B, L, D) + 0.0 * glove_loss
    return (emb, emb_glove, glove_loss)


# speedup vs baseline: 1.5963x; 1.0979x over previous
import functools

import jax
import jax.numpy as jnp
from jax import lax
from jax.experimental import pallas as pl
from jax.experimental.pallas import tpu as pltpu
from jax.experimental.pallas import tpu_sc as plsc

NTOKEN = 100000
D = 64
B = 16384
L = 50
N = B * L
NC = 2
NS = 16
NW = NC * NS
CHUNK = 128
PAIRS = CHUNK // 2
PER_W = N // NW
NSTEPS = PER_W // CHUNK
LANES = 16

_mesh = plsc.VectorSubcoreMesh(core_axis_name="c", subcore_axis_name="s")


@functools.partial(
    pl.kernel,
    out_type=(
        jax.ShapeDtypeStruct((N // 2, 128), jnp.float32),
        jax.ShapeDtypeStruct((N // 2, 128), jnp.float32),
        jax.ShapeDtypeStruct((NW * LANES,), jnp.float32),
    ),
    mesh=_mesh,
    compiler_params=pltpu.CompilerParams(use_tc_tiling_on_sc=False),
    scratch_types=[
        pltpu.VMEM((NSTEPS, CHUNK), jnp.int32),
        [pltpu.VMEM((CHUNK, D), jnp.float32)] * 2,
        [pltpu.VMEM((CHUNK, D), jnp.float32)] * 2,
        [pltpu.VMEM((PAIRS, 128), jnp.float32)] * 2,
        [pltpu.VMEM((PAIRS, 128), jnp.float32)] * 2,
        pltpu.VMEM((LANES,), jnp.float32),
        [pltpu.SemaphoreType.DMA] * 2,
        [pltpu.SemaphoreType.DMA] * 2,
    ],
)
def _sc_gather(idx_hbm, enc_hbm, glv_hbm, out_e, out_g, out_p,
               idx_v, e_slots, g_slots, es_slots, gs_slots, acc_v, gsems, wsems):
    wid = lax.axis_index("s") * NC + lax.axis_index("c")
    row0 = wid * PER_W
    q0 = row0 // 2
    pltpu.sync_copy(idx_hbm.at[pl.ds(wid * NSTEPS, NSTEPS)], idx_v)

    def fire_gather(k, s):
        pltpu.async_copy(enc_hbm.at[idx_v.at[k]], e_slots[s], gsems[s])
        pltpu.async_copy(glv_hbm.at[idx_v.at[k]], g_slots[s], gsems[s])

    def wait_gather(s):
        pltpu.make_async_copy(enc_hbm.at[idx_v.at[0]], e_slots[s], gsems[s]).wait()
        pltpu.make_async_copy(glv_hbm.at[idx_v.at[0]], g_slots[s], gsems[s]).wait()

    def fire_write(k, s):
        dst = pl.ds(q0 + k * PAIRS, PAIRS)
        pltpu.async_copy(es_slots[s], out_e.at[dst], wsems[s])
        pltpu.async_copy(gs_slots[s], out_g.at[dst], wsems[s])

    def wait_write(s):
        pltpu.make_async_copy(es_slots[s], out_e.at[pl.ds(0, PAIRS)], wsems[s]).wait()
        pltpu.make_async_copy(gs_slots[s], out_g.at[pl.ds(0, PAIRS)], wsems[s]).wait()

    def compute(s, accs):
        e_v, g_v = e_slots[s], g_slots[s]
        es_v, gs_v = es_slots[s], gs_slots[s]

        def pair_body(p, accs):
            a0, a1, a2, a3 = accs
            i0 = 2 * p
            i1 = i0 + 1
            for j in range(4):
                ve = e_v[i0, pl.ds(16 * j, LANES)]
                vg = g_v[i0, pl.ds(16 * j, LANES)]
                es_v[p, pl.ds(16 * j, LANES)] = ve
                gs_v[p, pl.ds(16 * j, LANES)] = vg
                d = ve - vg
                if j == 0:
                    a0 += d * d
                elif j == 1:
                    a1 += d * d
                elif j == 2:
                    a2 += d * d
                else:
                    a3 += d * d
            for j in range(4):
                ve = e_v[i1, pl.ds(16 * j, LANES)]
                vg = g_v[i1, pl.ds(16 * j, LANES)]
                es_v[p, pl.ds(64 + 16 * j, LANES)] = ve
                gs_v[p, pl.ds(64 + 16 * j, LANES)] = vg
                d = ve - vg
                if j == 0:
                    a0 += d * d
                elif j == 1:
                    a1 += d * d
                elif j == 2:
                    a2 += d * d
                else:
                    a3 += d * d
            return (a0, a1, a2, a3)

        return lax.fori_loop(0, PAIRS, pair_body, accs)

    def service(k, s, accs, *, first=False, last=False):
        wait_gather(s)
        if not first:
            wait_write(s)
        accs = compute(s, accs)
        fire_write(k, s)
        if not last:
            fire_gather(k + 2, s)
        return accs

    zero = jnp.zeros((LANES,), jnp.float32)
    accs = (zero, zero, zero, zero)

    fire_gather(0, 0)
    fire_gather(1, 1)
    accs = service(0, 0, accs, first=True)
    accs = service(1, 1, accs, first=True)

    def group_body(g, accs):
        k = 2 * g
        accs = service(k, 0, accs)
        accs = service(k + 1, 1, accs)
        return accs

    accs = lax.fori_loop(1, NSTEPS // 2 - 1, group_body, accs)

    k = NSTEPS - 2
    accs = service(k, 0, accs, last=True)
    accs = service(k + 1, 1, accs, last=True)
    wait_write(0)
    wait_write(1)

    a0, a1, a2, a3 = accs
    acc_v[...] = (a0 + a1) + (a2 + a3)
    pltpu.sync_copy(acc_v, out_p.at[pl.ds(wid * LANES, LANES)])


def _tc_sum_body(p_ref, o_ref):
    o_ref[0, 0] = jnp.sum(p_ref[...]) * jnp.float32(1.0 / (N * D))


_tc_sum = pl.pallas_call(
    _tc_sum_body,
    out_shape=jax.ShapeDtypeStruct((1, 1), jnp.float32),
    out_specs=pl.BlockSpec(memory_space=pltpu.SMEM),
)


def kernel(input, encoder_weight, glove_weight):
    idx = input.reshape(N // CHUNK, CHUNK).astype(jnp.int32)
    emb2, emb_glove2, parts = _sc_gather(idx, encoder_weight, glove_weight)
    glove_loss = _tc_sum(parts.reshape(4, 128))[0, 0]
    return (emb2.reshape(B, L, D), emb_glove2.reshape(B, L, D), glove_loss)
